# Initial kernel scaffold; baseline (speedup 1.0000x reference)
import jax
import jax.numpy as jnp
from jax.experimental import pallas as pl


def kernel(x, edge_index, W1, b1, W2, b2, Wl, bl):
    raise NotImplementedError("write your pallas kernel here")


# speedup vs baseline: 3.7540x; 3.7540x over previous
import functools

import jax
import jax.numpy as jnp
from jax import lax
from jax.experimental import pallas as pl
from jax.experimental.pallas import tpu as pltpu
from jax.experimental.pallas import tpu_sc as plsc

NC = 2
NS = 16
NW = NC * NS


def _sc_mesh():
    return plsc.VectorSubcoreMesh(
        core_axis_name="c", subcore_axis_name="s", num_cores=NC, num_subcores=NS
    )


def _make_deg_kernel(N, E, K):
    EB = E // NW
    NCH = EB // K
    RPT = N // NS

    @functools.partial(
        pl.kernel,
        out_type=jax.ShapeDtypeStruct((NC, N, 128), jnp.float32),
        mesh=_sc_mesh(),
        scratch_types=[
            pltpu.VMEM((1, K), jnp.int32),
            pltpu.VMEM((1, K), jnp.int32),
            pltpu.VMEM((K, 128), jnp.float32),
            pltpu.VMEM((K, 128), jnp.float32),
            pltpu.VMEM_SHARED((N, 128), jnp.float32),
        ],
    )
    def deg_kernel(src_h, dst_h, ones_s_h, ones_d_h, zer_h, dg_out,
                   sidx, didx, ones_s, ones_d, dg_sh):
        c = lax.axis_index("c")
        s = lax.axis_index("s")
        wid = s * NC + c
        r0 = s * RPT
        pltpu.sync_copy(zer_h, dg_sh.at[pl.ds(r0, RPT)])
        pltpu.sync_copy(ones_s_h, ones_s)
        pltpu.sync_copy(ones_d_h, ones_d)
        plsc.subcore_barrier()
        base = wid * EB

        def body(i, carry):
            off = pl.multiple_of(base + i * K, 8)
            pltpu.sync_copy(src_h.at[pl.ds(off, K)], sidx.at[0])
            pltpu.sync_copy(dst_h.at[pl.ds(off, K)], didx.at[0])
            pltpu.sync_copy(ones_s, dg_sh.at[sidx.at[0]], add=True)
            pltpu.sync_copy(ones_d, dg_sh.at[didx.at[0]], add=True)
            return carry

        lax.fori_loop(0, NCH, body, 0)
        plsc.subcore_barrier()
        pltpu.sync_copy(dg_sh.at[pl.ds(r0, RPT)], dg_out.at[c, pl.ds(r0, RPT)])

    return deg_kernel


def _make_agg_kernel(N, E, D, K):
    EB = E // NW
    NCH = EB // K
    RPT = N // NS

    @functools.partial(
        pl.kernel,
        out_type=jax.ShapeDtypeStruct((NC, N, D), jnp.float32),
        mesh=_sc_mesh(),
        scratch_types=[
            pltpu.VMEM((1, K), jnp.int32),
            pltpu.VMEM((1, K), jnp.int32),
            pltpu.VMEM((K, D), jnp.float32),
            pltpu.VMEM_SHARED((N, D), jnp.float32),
            pltpu.SemaphoreType.DMA,
        ],
    )
    def agg_kernel(hs_h, src_h, dst_h, zer_h, out_h,
                   sidx, didx, rows_v, agg_sh, sem):
        c = lax.axis_index("c")
        s = lax.axis_index("s")
        wid = s * NC + c
        r0 = s * RPT
        pltpu.sync_copy(zer_h, agg_sh.at[pl.ds(r0, RPT)])
        plsc.subcore_barrier()
        base = wid * EB

        def body(i, carry):
            off = pl.multiple_of(base + i * K, 8)
            pltpu.sync_copy(src_h.at[pl.ds(off, K)], sidx.at[0])
            pltpu.sync_copy(dst_h.at[pl.ds(off, K)], didx.at[0])
            pltpu.async_copy(hs_h.at[sidx.at[0]], rows_v, sem).wait()
            pltpu.sync_copy(rows_v, agg_sh.at[didx.at[0]], add=True)
            return carry

        lax.fori_loop(0, NCH, body, 0)
        plsc.subcore_barrier()
        pltpu.sync_copy(agg_sh.at[pl.ds(r0, RPT)], out_h.at[c, pl.ds(r0, RPT)])

    return agg_kernel


def _norms_from_deg(dg_block):
    d = dg_block[0] + dg_block[1]
    n_src = lax.rsqrt(jnp.maximum(d[:, 0:1], 1.0))
    n_dst = lax.rsqrt(jnp.maximum(d[:, 64:65], 1.0))
    return n_src, n_dst


def _tc_pre(x, W1, degp, BN):
    N, D = x.shape

    def body(x_ref, w_ref, dg_ref, o_ref):
        norm, _ = _norms_from_deg(dg_ref[...])
        o_ref[...] = (
            jnp.dot(x_ref[...], w_ref[...], preferred_element_type=jnp.float32)
            * norm
        )

    return pl.pallas_call(
        body,
        grid=(N // BN,),
        in_specs=[
            pl.BlockSpec((BN, D), lambda i: (i, 0)),
            pl.BlockSpec((D, D), lambda i: (0, 0)),
            pl.BlockSpec((2, BN, 128), lambda i: (0, i, 0)),
        ],
        out_specs=pl.BlockSpec((BN, D), lambda i: (i, 0)),
        out_shape=jax.ShapeDtypeStruct((N, D), jnp.float32),
    )(x, W1, degp)


def _tc_mid(aggp, degp, b1, W2, BN):
    _, N, D = aggp.shape

    def body(ag_ref, dg_ref, b_ref, w_ref, o_ref):
        agg = ag_ref[0] + ag_ref[1]
        n_src, n_dst = _norms_from_deg(dg_ref[...])
        h = jnp.maximum(agg * n_dst + b_ref[...], 0.0)
        o_ref[...] = (
            jnp.dot(h, w_ref[...], preferred_element_type=jnp.float32)
            * n_src
        )

    return pl.pallas_call(
        body,
        grid=(N // BN,),
        in_specs=[
            pl.BlockSpec((2, BN, D), lambda i: (0, i, 0)),
            pl.BlockSpec((2, BN, 128), lambda i: (0, i, 0)),
            pl.BlockSpec((1, D), lambda i: (0, 0)),
            pl.BlockSpec((D, D), lambda i: (0, 0)),
        ],
        out_specs=pl.BlockSpec((BN, D), lambda i: (i, 0)),
        out_shape=jax.ShapeDtypeStruct((N, D), jnp.float32),
    )(aggp, degp, b1, W2)


def _tc_final(aggp, degp, b2, Wl, bl, BN, Nvalid):
    _, N, D = aggp.shape
    Dout = Wl.shape[1]
    nsteps = N // BN

    def body(ag_ref, dg_ref, b_ref, wl_ref, bl_ref, o_ref, acc_ref):
        i = pl.program_id(0)
        agg = ag_ref[0] + ag_ref[1]
        _, n_dst = _norms_from_deg(dg_ref[...])
        h = jnp.maximum(agg * n_dst + b_ref[...], 0.0)
        row = i * BN + lax.broadcasted_iota(jnp.int32, (BN, 1), 0)
        h = jnp.where(row < Nvalid, h, 0.0)
        part = jnp.sum(h, axis=0, keepdims=True)

        @pl.when(i == 0)
        def _():
            acc_ref[...] = part

        @pl.when(i > 0)
        def _():
            acc_ref[...] += part

        @pl.when(i == nsteps - 1)
        def _():
            m = acc_ref[...] * jnp.float32(1.0 / Nvalid)
            o_ref[...] = jnp.maximum(
                jnp.dot(m, wl_ref[...], preferred_element_type=jnp.float32)
                + bl_ref[...],
                0.0,
            )

    return pl.pallas_call(
        body,
        grid=(nsteps,),
        in_specs=[
            pl.BlockSpec((2, BN, D), lambda i: (0, i, 0)),
            pl.BlockSpec((2, BN, 128), lambda i: (0, i, 0)),
            pl.BlockSpec((1, D), lambda i: (0, 0)),
            pl.BlockSpec((D, Dout), lambda i: (0, 0)),
            pl.BlockSpec((1, Dout), lambda i: (0, 0)),
        ],
        out_specs=pl.BlockSpec((1, Dout), lambda i: (0, 0)),
        out_shape=jax.ShapeDtypeStruct((1, Dout), jnp.float32),
        scratch_shapes=[pltpu.VMEM((1, D), jnp.float32)],
    )(aggp, degp, b2, Wl, bl)


def kernel(x, edge_index, W1, b1, W2, b2, Wl, bl):
    N, D = x.shape
    E = edge_index.shape[1]
    K = 80

    NP = -(-N // (NS * 8)) * (NS * 8)
    BN = NP // 16

    src = edge_index[0]
    dst = edge_index[1]
    xp = jnp.pad(x, ((0, NP - N), (0, 0)))
    col = jnp.arange(128) < 64
    ones_s = jnp.where(col, 1.0, 0.0)[None, :] * jnp.ones((K, 1), jnp.float32)
    ones_d = jnp.where(col, 0.0, 1.0)[None, :] * jnp.ones((K, 1), jnp.float32)
    zeros_d = jnp.zeros((NP // NS, 128), jnp.float32)
    zeros_a = jnp.zeros((NP // NS, D), jnp.float32)

    deg_kernel = _make_deg_kernel(NP, E, K)
    agg_kernel = _make_agg_kernel(NP, E, D, K)

    degp = deg_kernel(src, dst, ones_s, ones_d, zeros_d)
    hs1 = _tc_pre(xp, W1, degp, BN)
    aggp1 = agg_kernel(hs1, src, dst, zeros_a)
    hs2 = _tc_mid(aggp1, degp, b1.reshape(1, -1), W2, BN)
    aggp2 = agg_kernel(hs2, src, dst, zeros_a)
    return _tc_final(aggp2, degp, b2.reshape(1, -1), Wl, bl.reshape(1, -1), BN, N)


# speedup vs baseline: 5.7890x; 1.5421x over previous
import functools

import jax
import jax.numpy as jnp
from jax import lax
from jax.experimental import pallas as pl
from jax.experimental.pallas import tpu as pltpu
from jax.experimental.pallas import tpu_sc as plsc

NC = 2
NS = 16
NW = NC * NS


def _sc_mesh():
    return plsc.VectorSubcoreMesh(
        core_axis_name="c", subcore_axis_name="s", num_cores=NC, num_subcores=NS
    )


def _make_deg_kernel(N, E, K):
    EB = E // NW
    NCH = EB // K
    RPT = N // NS

    @functools.partial(
        pl.kernel,
        out_type=jax.ShapeDtypeStruct((NC, N, 128), jnp.float32),
        mesh=_sc_mesh(),
        scratch_types=[
            pltpu.VMEM((1, K), jnp.int32),
            pltpu.VMEM((1, K), jnp.int32),
            pltpu.VMEM((K, 128), jnp.float32),
            pltpu.VMEM((K, 128), jnp.float32),
            pltpu.VMEM_SHARED((N, 128), jnp.float32),
        ],
    )
    def deg_kernel(src_h, dst_h, ones_s_h, ones_d_h, zer_h, dg_out,
                   sidx, didx, ones_s, ones_d, dg_sh):
        c = lax.axis_index("c")
        s = lax.axis_index("s")
        wid = s * NC + c
        r0 = s * RPT
        pltpu.sync_copy(zer_h, dg_sh.at[pl.ds(r0, RPT)])
        pltpu.sync_copy(ones_s_h, ones_s)
        pltpu.sync_copy(ones_d_h, ones_d)
        plsc.subcore_barrier()
        base = wid * EB

        def body(i, carry):
            off = pl.multiple_of(base + i * K, 8)
            pltpu.sync_copy(src_h.at[pl.ds(off, K)], sidx.at[0])
            pltpu.sync_copy(dst_h.at[pl.ds(off, K)], didx.at[0])
            pltpu.sync_copy(ones_s, dg_sh.at[sidx.at[0]], add=True)
            pltpu.sync_copy(ones_d, dg_sh.at[didx.at[0]], add=True)
            return carry

        lax.fori_loop(0, NCH, body, 0)
        plsc.subcore_barrier()
        pltpu.sync_copy(dg_sh.at[pl.ds(r0, RPT)], dg_out.at[c, pl.ds(r0, RPT)])

    return deg_kernel


def _make_agg_kernel(N, E, D, K):
    EB = E // NW
    NCH = EB // K
    NB = NCH // 2
    RPT = N // NS

    @functools.partial(
        pl.kernel,
        out_type=jax.ShapeDtypeStruct((NC, N, D), jnp.float32),
        mesh=_sc_mesh(),
        scratch_types=[
            pltpu.VMEM((2, K), jnp.int32),
            pltpu.VMEM((2, K), jnp.int32),
            pltpu.VMEM((K, D), jnp.float32),
            pltpu.VMEM((K, D), jnp.float32),
            pltpu.VMEM_SHARED((N, D), jnp.float32),
            pltpu.SemaphoreType.DMA,
            pltpu.SemaphoreType.DMA,
            pltpu.SemaphoreType.DMA,
            pltpu.SemaphoreType.DMA,
        ],
    )
    def agg_kernel(hs_h, src_h, dst_h, zer_h, out_h,
                   sidx, didx, rows0, rows1, agg_sh,
                   sem_i0, sem_i1, sem_g0, sem_g1):
        c = lax.axis_index("c")
        s = lax.axis_index("s")
        wid = s * NC + c
        r0 = s * RPT
        pltpu.sync_copy(zer_h, agg_sh.at[pl.ds(r0, RPT)])
        plsc.subcore_barrier()
        base = wid * EB

        def fire_idx(i, slot, sem):
            off = pl.multiple_of(base + i * K, 8)
            pltpu.async_copy(src_h.at[pl.ds(off, K)], sidx.at[slot], sem)
            pltpu.async_copy(dst_h.at[pl.ds(off, K)], didx.at[slot], sem)

        def wait_idx(slot, sem):
            pltpu.make_async_copy(src_h.at[pl.ds(0, K)], sidx.at[slot], sem).wait()
            pltpu.make_async_copy(dst_h.at[pl.ds(0, K)], didx.at[slot], sem).wait()

        def fire_gather(slot_rows, slot, sem):
            pltpu.async_copy(hs_h.at[sidx.at[slot]], slot_rows, sem)

        def wait_gather(slot_rows, slot, sem):
            pltpu.make_async_copy(hs_h.at[sidx.at[slot]], slot_rows, sem).wait()

        fire_idx(0, 0, sem_i0)
        fire_idx(1, 1, sem_i1)
        wait_idx(0, sem_i0)
        fire_gather(rows0, 0, sem_g0)

        def body(t, carry):
            wait_idx(1, sem_i1)
            fire_gather(rows1, 1, sem_g1)
            wait_gather(rows0, 0, sem_g0)
            pltpu.sync_copy(rows0, agg_sh.at[didx.at[0]], add=True)
            fire_idx(2 * t + 2, 0, sem_i0)
            wait_idx(0, sem_i0)
            fire_gather(rows0, 0, sem_g0)
            wait_gather(rows1, 1, sem_g1)
            pltpu.sync_copy(rows1, agg_sh.at[didx.at[1]], add=True)

            @pl.when(t < NB - 1)
            def _():
                fire_idx(2 * t + 3, 1, sem_i1)

            return carry

        lax.fori_loop(0, NB, body, 0)
        wait_gather(rows0, 0, sem_g0)
        pltpu.sync_copy(rows0, agg_sh.at[didx.at[0]], add=True)
        plsc.subcore_barrier()
        pltpu.sync_copy(agg_sh.at[pl.ds(r0, RPT)], out_h.at[c, pl.ds(r0, RPT)])

    return agg_kernel


def _norms_from_deg(dg_block):
    d = dg_block[0] + dg_block[1]
    n_src = lax.rsqrt(jnp.maximum(d[:, 0:1], 1.0))
    n_dst = lax.rsqrt(jnp.maximum(d[:, 64:65], 1.0))
    return n_src, n_dst


def _tc_pre(x, W1, degp, BN):
    N, D = x.shape

    def body(x_ref, w_ref, dg_ref, o_ref):
        norm, _ = _norms_from_deg(dg_ref[...])
        o_ref[...] = (
            jnp.dot(x_ref[...], w_ref[...], preferred_element_type=jnp.float32)
            * norm
        )

    return pl.pallas_call(
        body,
        grid=(N // BN,),
        in_specs=[
            pl.BlockSpec((BN, D), lambda i: (i, 0)),
            pl.BlockSpec((D, D), lambda i: (0, 0)),
            pl.BlockSpec((2, BN, 128), lambda i: (0, i, 0)),
        ],
        out_specs=pl.BlockSpec((BN, D), lambda i: (i, 0)),
        out_shape=jax.ShapeDtypeStruct((N, D), jnp.float32),
    )(x, W1, degp)


def _tc_mid(aggp, degp, b1, W2, BN):
    _, N, D = aggp.shape

    def body(ag_ref, dg_ref, b_ref, w_ref, o_ref):
        agg = ag_ref[0] + ag_ref[1]
        n_src, n_dst = _norms_from_deg(dg_ref[...])
        h = jnp.maximum(agg * n_dst + b_ref[...], 0.0)
        o_ref[...] = (
            jnp.dot(h, w_ref[...], preferred_element_type=jnp.float32)
            * n_src
        )

    return pl.pallas_call(
        body,
        grid=(N // BN,),
        in_specs=[
            pl.BlockSpec((2, BN, D), lambda i: (0, i, 0)),
            pl.BlockSpec((2, BN, 128), lambda i: (0, i, 0)),
            pl.BlockSpec((1, D), lambda i: (0, 0)),
            pl.BlockSpec((D, D), lambda i: (0, 0)),
        ],
        out_specs=pl.BlockSpec((BN, D), lambda i: (i, 0)),
        out_shape=jax.ShapeDtypeStruct((N, D), jnp.float32),
    )(aggp, degp, b1, W2)


def _tc_final(aggp, degp, b2, Wl, bl, BN, Nvalid):
    _, N, D = aggp.shape
    Dout = Wl.shape[1]
    nsteps = N // BN

    def body(ag_ref, dg_ref, b_ref, wl_ref, bl_ref, o_ref, acc_ref):
        i = pl.program_id(0)
        agg = ag_ref[0] + ag_ref[1]
        _, n_dst = _norms_from_deg(dg_ref[...])
        h = jnp.maximum(agg * n_dst + b_ref[...], 0.0)
        row = i * BN + lax.broadcasted_iota(jnp.int32, (BN, 1), 0)
        h = jnp.where(row < Nvalid, h, 0.0)
        part = jnp.sum(h, axis=0, keepdims=True)

        @pl.when(i == 0)
        def _():
            acc_ref[...] = part

        @pl.when(i > 0)
        def _():
            acc_ref[...] += part

        @pl.when(i == nsteps - 1)
        def _():
            m = acc_ref[...] * jnp.float32(1.0 / Nvalid)
            o_ref[...] = jnp.maximum(
                jnp.dot(m, wl_ref[...], preferred_element_type=jnp.float32)
                + bl_ref[...],
                0.0,
            )

    return pl.pallas_call(
        body,
        grid=(nsteps,),
        in_specs=[
            pl.BlockSpec((2, BN, D), lambda i: (0, i, 0)),
            pl.BlockSpec((2, BN, 128), lambda i: (0, i, 0)),
            pl.BlockSpec((1, D), lambda i: (0, 0)),
            pl.BlockSpec((D, Dout), lambda i: (0, 0)),
            pl.BlockSpec((1, Dout), lambda i: (0, 0)),
        ],
        out_specs=pl.BlockSpec((1, Dout), lambda i: (0, 0)),
        out_shape=jax.ShapeDtypeStruct((1, Dout), jnp.float32),
        scratch_shapes=[pltpu.VMEM((1, D), jnp.float32)],
    )(aggp, degp, b2, Wl, bl)


def kernel(x, edge_index, W1, b1, W2, b2, Wl, bl):
    N, D = x.shape
    E = edge_index.shape[1]
    K = 80

    NP = -(-N // (NS * 8)) * (NS * 8)
    BN = NP // 16

    src = edge_index[0]
    dst = edge_index[1]
    xp = jnp.pad(x, ((0, NP - N), (0, 0)))
    col = jnp.arange(128) < 64
    ones_s = jnp.where(col, 1.0, 0.0)[None, :] * jnp.ones((K, 1), jnp.float32)
    ones_d = jnp.where(col, 0.0, 1.0)[None, :] * jnp.ones((K, 1), jnp.float32)
    zeros_d = jnp.zeros((NP // NS, 128), jnp.float32)
    zeros_a = jnp.zeros((NP // NS, D), jnp.float32)

    deg_kernel = _make_deg_kernel(NP, E, K)
    agg_kernel = _make_agg_kernel(NP, E, D, K)

    degp = deg_kernel(src, dst, ones_s, ones_d, zeros_d)
    hs1 = _tc_pre(xp, W1, degp, BN)
    aggp1 = agg_kernel(hs1, src, dst, zeros_a)
    hs2 = _tc_mid(aggp1, degp, b1.reshape(1, -1), W2, BN)
    aggp2 = agg_kernel(hs2, src, dst, zeros_a)
    return _tc_final(aggp2, degp, b2.reshape(1, -1), Wl, bl.reshape(1, -1), BN, N)


# speedup vs baseline: 6.9907x; 1.2076x over previous
import functools

import jax
import jax.numpy as jnp
from jax import lax
from jax.experimental import pallas as pl
from jax.experimental.pallas import tpu as pltpu
from jax.experimental.pallas import tpu_sc as plsc

NC = 2
NS = 16
NW = NC * NS


def _sc_mesh():
    return plsc.VectorSubcoreMesh(
        core_axis_name="c", subcore_axis_name="s", num_cores=NC, num_subcores=NS
    )


def _make_deg_kernel(N, E, K):
    EB = E // NW
    NCH = EB // K
    NB = NCH // 2
    assert NCH % 2 == 1
    RPT = N // NS

    @functools.partial(
        pl.kernel,
        out_type=jax.ShapeDtypeStruct((NC, N, 128), jnp.float32),
        mesh=_sc_mesh(),
        scratch_types=[
            pltpu.VMEM((2, K), jnp.int32),
            pltpu.VMEM((2, K), jnp.int32),
            pltpu.VMEM((K, 128), jnp.float32),
            pltpu.VMEM((K, 128), jnp.float32),
            pltpu.VMEM_SHARED((N, 128), jnp.float32),
            pltpu.SemaphoreType.DMA,
            pltpu.SemaphoreType.DMA,
        ],
    )
    def deg_kernel(src_h, dst_h, ones_s_h, ones_d_h, zer_h, dg_out,
                   sidx, didx, ones_s, ones_d, dg_sh, sem_i0, sem_i1):
        c = lax.axis_index("c")
        s = lax.axis_index("s")
        wid = s * NC + c
        r0 = s * RPT
        pltpu.sync_copy(zer_h, dg_sh.at[pl.ds(r0, RPT)])
        pltpu.sync_copy(ones_s_h, ones_s)
        pltpu.sync_copy(ones_d_h, ones_d)
        plsc.subcore_barrier()
        base = wid * EB

        def fire_idx(i, slot, sem):
            off = pl.multiple_of(base + i * K, 8)
            pltpu.async_copy(src_h.at[pl.ds(off, K)], sidx.at[slot], sem)
            pltpu.async_copy(dst_h.at[pl.ds(off, K)], didx.at[slot], sem)

        def wait_idx(slot, sem):
            pltpu.make_async_copy(src_h.at[pl.ds(0, K)], sidx.at[slot], sem).wait()
            pltpu.make_async_copy(dst_h.at[pl.ds(0, K)], didx.at[slot], sem).wait()

        fire_idx(0, 0, sem_i0)
        fire_idx(1, 1, sem_i1)

        def body(t, carry):
            wait_idx(0, sem_i0)
            pltpu.sync_copy(ones_s, dg_sh.at[sidx.at[0]], add=True)
            pltpu.sync_copy(ones_d, dg_sh.at[didx.at[0]], add=True)
            fire_idx(2 * t + 2, 0, sem_i0)
            wait_idx(1, sem_i1)
            pltpu.sync_copy(ones_s, dg_sh.at[sidx.at[1]], add=True)
            pltpu.sync_copy(ones_d, dg_sh.at[didx.at[1]], add=True)

            @pl.when(t < NB - 1)
            def _():
                fire_idx(2 * t + 3, 1, sem_i1)

            return carry

        lax.fori_loop(0, NB, body, 0)
        wait_idx(0, sem_i0)
        pltpu.sync_copy(ones_s, dg_sh.at[sidx.at[0]], add=True)
        pltpu.sync_copy(ones_d, dg_sh.at[didx.at[0]], add=True)
        plsc.subcore_barrier()
        pltpu.sync_copy(dg_sh.at[pl.ds(r0, RPT)], dg_out.at[c, pl.ds(r0, RPT)])

    return deg_kernel


def _make_agg_kernel(N, E, D, K):
    EB = E // NW
    NCH = EB // K
    NB = NCH // 2
    RPT = N // NS

    @functools.partial(
        pl.kernel,
        out_type=jax.ShapeDtypeStruct((NC, N, D), jnp.float32),
        mesh=_sc_mesh(),
        scratch_types=[
            pltpu.VMEM((2, K), jnp.int32),
            pltpu.VMEM((2, K), jnp.int32),
            pltpu.VMEM((K, D), jnp.float32),
            pltpu.VMEM((K, D), jnp.float32),
            pltpu.VMEM_SHARED((N, D), jnp.float32),
            pltpu.SemaphoreType.DMA,
            pltpu.SemaphoreType.DMA,
            pltpu.SemaphoreType.DMA,
            pltpu.SemaphoreType.DMA,
        ],
    )
    def agg_kernel(hs_h, src_h, dst_h, zer_h, out_h,
                   sidx, didx, rows0, rows1, agg_sh,
                   sem_i0, sem_i1, sem_g0, sem_g1):
        c = lax.axis_index("c")
        s = lax.axis_index("s")
        wid = s * NC + c
        r0 = s * RPT
        pltpu.sync_copy(zer_h, agg_sh.at[pl.ds(r0, RPT)])
        plsc.subcore_barrier()
        base = wid * EB

        def fire_idx(i, slot, sem):
            off = pl.multiple_of(base + i * K, 8)
            pltpu.async_copy(src_h.at[pl.ds(off, K)], sidx.at[slot], sem)
            pltpu.async_copy(dst_h.at[pl.ds(off, K)], didx.at[slot], sem)

        def wait_idx(slot, sem):
            pltpu.make_async_copy(src_h.at[pl.ds(0, K)], sidx.at[slot], sem).wait()
            pltpu.make_async_copy(dst_h.at[pl.ds(0, K)], didx.at[slot], sem).wait()

        def fire_gather(slot_rows, slot, sem):
            pltpu.async_copy(hs_h.at[sidx.at[slot]], slot_rows, sem)

        def wait_gather(slot_rows, slot, sem):
            pltpu.make_async_copy(hs_h.at[sidx.at[slot]], slot_rows, sem).wait()

        fire_idx(0, 0, sem_i0)
        fire_idx(1, 1, sem_i1)
        wait_idx(0, sem_i0)
        fire_gather(rows0, 0, sem_g0)

        def body(t, carry):
            wait_idx(1, sem_i1)
            fire_gather(rows1, 1, sem_g1)
            wait_gather(rows0, 0, sem_g0)
            pltpu.sync_copy(rows0, agg_sh.at[didx.at[0]], add=True)
            fire_idx(2 * t + 2, 0, sem_i0)
            wait_idx(0, sem_i0)
            fire_gather(rows0, 0, sem_g0)
            wait_gather(rows1, 1, sem_g1)
            pltpu.sync_copy(rows1, agg_sh.at[didx.at[1]], add=True)

            @pl.when(t < NB - 1)
            def _():
                fire_idx(2 * t + 3, 1, sem_i1)

            return carry

        lax.fori_loop(0, NB, body, 0)
        wait_gather(rows0, 0, sem_g0)
        pltpu.sync_copy(rows0, agg_sh.at[didx.at[0]], add=True)
        plsc.subcore_barrier()
        pltpu.sync_copy(agg_sh.at[pl.ds(r0, RPT)], out_h.at[c, pl.ds(r0, RPT)])

    return agg_kernel


def _norms_from_deg(dg_block):
    d = dg_block[0] + dg_block[1]
    n_src = lax.rsqrt(jnp.maximum(d[:, 0:1], 1.0))
    n_dst = lax.rsqrt(jnp.maximum(d[:, 64:65], 1.0))
    return n_src, n_dst


def _tc_pre(x, W1, degp, BN):
    N, D = x.shape

    def body(x_ref, w_ref, dg_ref, o_ref):
        norm, _ = _norms_from_deg(dg_ref[...])
        o_ref[...] = (
            jnp.dot(x_ref[...], w_ref[...], preferred_element_type=jnp.float32)
            * norm
        )

    return pl.pallas_call(
        body,
        grid=(N // BN,),
        in_specs=[
            pl.BlockSpec((BN, D), lambda i: (i, 0)),
            pl.BlockSpec((D, D), lambda i: (0, 0)),
            pl.BlockSpec((2, BN, 128), lambda i: (0, i, 0)),
        ],
        out_specs=pl.BlockSpec((BN, D), lambda i: (i, 0)),
        out_shape=jax.ShapeDtypeStruct((N, D), jnp.float32),
    )(x, W1, degp)


def _tc_mid(aggp, degp, b1, W2, BN):
    _, N, D = aggp.shape

    def body(ag_ref, dg_ref, b_ref, w_ref, o_ref):
        agg = ag_ref[0] + ag_ref[1]
        n_src, n_dst = _norms_from_deg(dg_ref[...])
        h = jnp.maximum(agg * n_dst + b_ref[...], 0.0)
        o_ref[...] = (
            jnp.dot(h, w_ref[...], preferred_element_type=jnp.float32)
            * n_src
        )

    return pl.pallas_call(
        body,
        grid=(N // BN,),
        in_specs=[
            pl.BlockSpec((2, BN, D), lambda i: (0, i, 0)),
            pl.BlockSpec((2, BN, 128), lambda i: (0, i, 0)),
            pl.BlockSpec((1, D), lambda i: (0, 0)),
            pl.BlockSpec((D, D), lambda i: (0, 0)),
        ],
        out_specs=pl.BlockSpec((BN, D), lambda i: (i, 0)),
        out_shape=jax.ShapeDtypeStruct((N, D), jnp.float32),
    )(aggp, degp, b1, W2)


def _tc_final(aggp, degp, b2, Wl, bl, BN, Nvalid):
    _, N, D = aggp.shape
    Dout = Wl.shape[1]
    nsteps = N // BN

    def body(ag_ref, dg_ref, b_ref, wl_ref, bl_ref, o_ref, acc_ref):
        i = pl.program_id(0)
        agg = ag_ref[0] + ag_ref[1]
        _, n_dst = _norms_from_deg(dg_ref[...])
        h = jnp.maximum(agg * n_dst + b_ref[...], 0.0)
        row = i * BN + lax.broadcasted_iota(jnp.int32, (BN, 1), 0)
        h = jnp.where(row < Nvalid, h, 0.0)
        part = jnp.sum(h, axis=0, keepdims=True)

        @pl.when(i == 0)
        def _():
            acc_ref[...] = part

        @pl.when(i > 0)
        def _():
            acc_ref[...] += part

        @pl.when(i == nsteps - 1)
        def _():
            m = acc_ref[...] * jnp.float32(1.0 / Nvalid)
            o_ref[...] = jnp.maximum(
                jnp.dot(m, wl_ref[...], preferred_element_type=jnp.float32)
                + bl_ref[...],
                0.0,
            )

    return pl.pallas_call(
        body,
        grid=(nsteps,),
        in_specs=[
            pl.BlockSpec((2, BN, D), lambda i: (0, i, 0)),
            pl.BlockSpec((2, BN, 128), lambda i: (0, i, 0)),
            pl.BlockSpec((1, D), lambda i: (0, 0)),
            pl.BlockSpec((D, Dout), lambda i: (0, 0)),
            pl.BlockSpec((1, Dout), lambda i: (0, 0)),
        ],
        out_specs=pl.BlockSpec((1, Dout), lambda i: (0, 0)),
        out_shape=jax.ShapeDtypeStruct((1, Dout), jnp.float32),
        scratch_shapes=[pltpu.VMEM((1, D), jnp.float32)],
    )(aggp, degp, b2, Wl, bl)


def kernel(x, edge_index, W1, b1, W2, b2, Wl, bl):
    N, D = x.shape
    E = edge_index.shape[1]
    K = 80

    NP = -(-N // (NS * 8)) * (NS * 8)
    BN = NP // 16

    src = edge_index[0]
    dst = edge_index[1]
    xp = jnp.pad(x, ((0, NP - N), (0, 0)))
    col = jnp.arange(128) < 64
    ones_s = jnp.where(col, 1.0, 0.0)[None, :] * jnp.ones((K, 1), jnp.float32)
    ones_d = jnp.where(col, 0.0, 1.0)[None, :] * jnp.ones((K, 1), jnp.float32)
    zeros_d = jnp.zeros((NP // NS, 128), jnp.float32)
    zeros_a = jnp.zeros((NP // NS, D), jnp.float32)

    deg_kernel = _make_deg_kernel(NP, E, K)
    agg_kernel = _make_agg_kernel(NP, E, D, K)

    degp = deg_kernel(src, dst, ones_s, ones_d, zeros_d)
    hs1 = _tc_pre(xp, W1, degp, BN)
    aggp1 = agg_kernel(hs1, src, dst, zeros_a)
    hs2 = _tc_mid(aggp1, degp, b1.reshape(1, -1), W2, BN)
    aggp2 = agg_kernel(hs2, src, dst, zeros_a)
    return _tc_final(aggp2, degp, b2.reshape(1, -1), Wl, bl.reshape(1, -1), BN, N)
